# baseline (speedup 1.0000x reference)
import functools

import jax
import jax.numpy as jnp
from jax import lax
from jax.experimental import pallas as pl
from jax.experimental.pallas import tpu as pltpu
from jax.experimental.pallas import tpu_sc as plsc

N = 320000
D = 128
NC, NS, L = 2, 16, 16
NW = NC * NS
RPW = N // NW
CH = 200
NCH = RPW // CH


def _sc_body(feat, ids, rec_s, aux, f0, f1, i0, i1, recs_v, aux_v, first_s,
             sf0, sf1, si0, si1):
    cid = lax.axis_index("c")
    sid = lax.axis_index("s")
    wid = cid * NS + sid
    base = wid * RPW

    zero = jnp.zeros((L,), jnp.float32)
    for k in range(2 * D // L):
        recs_v[pl.ds(k * L, L)] = zero
    for k in range(9):
        aux_v[pl.ds(k * L, L)] = zero
    first_s[0] = 1

    def start(c, fb, ib, semf, semi):
        r0 = base + c * CH
        pltpu.make_async_copy(feat.at[pl.ds(r0 * D, CH * D)], fb, semf).start()
        pltpu.make_async_copy(ids.at[pl.ds(r0, CH)], ib.at[pl.ds(0, CH)],
                              semi).start()

    def wait(fb, ib, semf, semi):
        pltpu.make_async_copy(feat.at[pl.ds(0, CH * D)], fb, semf).wait()
        pltpu.make_async_copy(ids.at[pl.ds(0, CH)], ib.at[pl.ds(0, CH)],
                              semi).wait()

    def finalize_run(pid, cntv, sxxv, s, to_first):
        @pl.when(to_first)
        def _():
            aux_v[pl.ds(0, L)] = pid.astype(jnp.float32) + zero
            aux_v[pl.ds(L, L)] = cntv
            aux_v[pl.ds(2 * L, L)] = sxxv
            for k in range(D // L):
                recs_v[pl.ds(k * L, L)] = s[k]
            first_s[0] = 0

        @pl.when(jnp.logical_not(to_first))
        def _():
            sv = s[0] * s[0]
            for k in range(1, D // L):
                sv = sv + s[k] * s[k]
            inv = 1.0 / cntv
            aux_v[pl.ds(6 * L, L)] = aux_v[pl.ds(6 * L, L)] + (
                sxxv * inv - sv * inv * inv)
            aux_v[pl.ds(7 * L, L)] = aux_v[pl.ds(7 * L, L)] + 1.0

    def process(fb, ib, carry):
        def row(r, carry):
            pid, cntv, sxxv = carry[0], carry[1], carry[2]
            s = carry[3:]
            rid = ib[pl.ds(r, L)][0]
            rb = r * D
            x = [fb[pl.ds(rb + k * L, L)] for k in range(D // L)]
            rowsq = x[0] * x[0]
            for k in range(1, D // L):
                rowsq = rowsq + x[k] * x[k]
            b = rid != pid

            @pl.when(b)
            def _():
                finalize_run(pid, cntv, sxxv, s, first_s[0] == 1)

            sn = tuple(jnp.where(b, x[k], s[k] + x[k]) for k in range(D // L))
            sxxn = jnp.where(b, rowsq, sxxv + rowsq)
            cntn = jnp.where(b, 1.0 + zero, cntv + 1.0)
            return (rid, cntn, sxxn) + sn

        return lax.fori_loop(0, CH, row, carry, unroll=4)

    carry0 = (jnp.int32(-1), zero, zero) + tuple(zero for _ in range(D // L))

    start(0, f0, i0, sf0, si0)

    def pair(p, carry):
        c0 = 2 * p
        start(c0 + 1, f1, i1, sf1, si1)
        wait(f0, i0, sf0, si0)
        carry = process(f0, i0, carry)
        start(c0 + 2, f0, i0, sf0, si0)
        wait(f1, i1, sf1, si1)
        carry = process(f1, i1, carry)
        return carry

    carry = lax.fori_loop(0, NCH // 2 - 1, pair, carry0)
    start(NCH - 1, f1, i1, sf1, si1)
    wait(f0, i0, sf0, si0)
    carry = process(f0, i0, carry)
    wait(f1, i1, sf1, si1)
    carry = process(f1, i1, carry)

    pid, cntv, sxxv = carry[0], carry[1], carry[2]
    s = carry[3:]
    isf = first_s[0] == 1

    @pl.when(isf)
    def _():
        aux_v[pl.ds(0, L)] = pid.astype(jnp.float32) + zero
        aux_v[pl.ds(L, L)] = cntv
        aux_v[pl.ds(2 * L, L)] = sxxv
        aux_v[pl.ds(8 * L, L)] = 1.0 + zero
        for k in range(D // L):
            recs_v[pl.ds(k * L, L)] = s[k]

    @pl.when(jnp.logical_not(isf))
    def _():
        aux_v[pl.ds(3 * L, L)] = pid.astype(jnp.float32) + zero
        aux_v[pl.ds(4 * L, L)] = cntv
        aux_v[pl.ds(5 * L, L)] = sxxv
        for k in range(D // L):
            recs_v[pl.ds(D + k * L, L)] = s[k]

    pltpu.sync_copy(recs_v, rec_s.at[wid])
    pltpu.sync_copy(aux_v, aux.at[wid])


def _merge_body(recs_ref, aux_ref, out_ref):
    def step(w, carry):
        c_id, c_cnt, c_sxx, c_s, loss, uniq = carry
        av = aux_ref[pl.ds(w, 1)][0]
        rv = recs_ref[pl.ds(w, 1)][0]
        f_id = jnp.sum(av[0]) * (1.0 / 16.0)
        f_cnt = jnp.sum(av[1]) * (1.0 / 16.0)
        f_sxx = jnp.sum(av[2])
        l_id = jnp.sum(av[3]) * (1.0 / 16.0)
        l_cnt = jnp.sum(av[4]) * (1.0 / 16.0)
        l_sxx = jnp.sum(av[5])
        int_loss = jnp.sum(av[6])
        int_uniq = jnp.sum(av[7]) * (1.0 / 16.0)
        single = jnp.sum(av[8]) * (1.0 / 16.0) > 0.5
        f_s = rv[0:1, :]
        l_s = rv[1:2, :]

        mrg = jnp.logical_and(c_cnt > 0.0, c_id == f_id)
        fin = jnp.logical_and(jnp.logical_not(mrg), c_cnt > 0.0)
        safe = jnp.maximum(c_cnt, 1.0)
        c_ssq = jnp.sum(c_s * c_s)
        loss = loss + jnp.where(fin, c_sxx / safe - c_ssq / (safe * safe), 0.0)
        uniq = uniq + jnp.where(fin, 1.0, 0.0)

        m = jnp.where(mrg, 1.0, 0.0)
        f_cnt2 = f_cnt + m * c_cnt
        f_sxx2 = f_sxx + m * c_sxx
        f_s2 = f_s + m * c_s
        f_ssq = jnp.sum(f_s2 * f_s2)
        ffin = f_sxx2 / f_cnt2 - f_ssq / (f_cnt2 * f_cnt2)
        loss = loss + jnp.where(single, 0.0, ffin + int_loss)
        uniq = uniq + jnp.where(single, 0.0, 1.0 + int_uniq)

        c_id = jnp.where(single, f_id, l_id)
        c_cnt = jnp.where(single, f_cnt2, l_cnt)
        c_sxx = jnp.where(single, f_sxx2, l_sxx)
        c_s = jnp.where(single, f_s2, l_s)
        return (c_id, c_cnt, c_sxx, c_s, loss, uniq)

    init = (jnp.float32(-1.0), jnp.float32(0.0), jnp.float32(0.0),
            jnp.zeros((1, D), jnp.float32), jnp.float32(0.0), jnp.float32(0.0))
    c_id, c_cnt, c_sxx, c_s, loss, uniq = lax.fori_loop(0, NW, step, init)
    loss = loss + c_sxx / c_cnt - jnp.sum(c_s * c_s) / (c_cnt * c_cnt)
    uniq = uniq + 1.0
    out_ref[...] = jnp.full((1, 1), loss / uniq, jnp.float32)


@jax.jit
def kernel(reid_feat, ids):
    sc_phase = pl.kernel(
        _sc_body,
        out_type=(
            jax.ShapeDtypeStruct((NW, 2 * D), jnp.float32),
            jax.ShapeDtypeStruct((NW, 9 * L), jnp.float32),
        ),
        mesh=plsc.VectorSubcoreMesh(core_axis_name="c", subcore_axis_name="s",
                                    num_cores=NC, num_subcores=NS),
        scratch_types=[
            pltpu.VMEM((CH * D,), jnp.float32),
            pltpu.VMEM((CH * D,), jnp.float32),
            pltpu.VMEM((CH + L,), jnp.int32),
            pltpu.VMEM((CH + L,), jnp.int32),
            pltpu.VMEM((2 * D,), jnp.float32),
            pltpu.VMEM((9 * L,), jnp.float32),
            pltpu.SMEM((1,), jnp.int32),
            pltpu.SemaphoreType.DMA,
            pltpu.SemaphoreType.DMA,
            pltpu.SemaphoreType.DMA,
            pltpu.SemaphoreType.DMA,
        ],
    )
    rec_s, aux = sc_phase(reid_feat.reshape(-1), ids.astype(jnp.int32))
    rec_s = rec_s.reshape(NW, 2, D)
    aux = aux.reshape(NW, 9, L)

    merged = pl.pallas_call(
        _merge_body,
        out_shape=jax.ShapeDtypeStruct((1, 1), jnp.float32),
        in_specs=[
            pl.BlockSpec(memory_space=pltpu.VMEM),
            pl.BlockSpec(memory_space=pltpu.VMEM),
        ],
        out_specs=pl.BlockSpec(memory_space=pltpu.VMEM),
    )(rec_s, aux)
    return merged[0, 0]


# speedup vs baseline: 1.9682x; 1.9682x over previous
import functools

import jax
import jax.numpy as jnp
from jax import lax
from jax.experimental import pallas as pl
from jax.experimental.pallas import tpu as pltpu
from jax.experimental.pallas import tpu_sc as plsc

N = 320000
D = 128
NC, NS, L = 2, 16, 16
NW = NC * NS
RPW = N // NW
CH = 400
NCH = RPW // CH
NG = CH // L


def _sc_body(feat, ids, rec_s, aux, f0, f1, i0, i1, st_v, recs_v, aux_v,
             first_s, pid_s, cnt_s, sf0, sf1, si0, si1):
    cid = lax.axis_index("c")
    sid = lax.axis_index("s")
    wid = cid * NS + sid
    base = wid * RPW

    zero = jnp.zeros((L,), jnp.float32)
    for k in range(2 * D // L):
        recs_v[pl.ds(k * L, L)] = zero
    for k in range(9):
        aux_v[pl.ds(k * L, L)] = zero
    for k in range(9):
        st_v[pl.ds(k * L, L)] = zero
    first_s[0] = 1
    pid_s[0] = -1
    cnt_s[0] = 0

    def start(c, fb, ib, semf, semi):
        r0 = base + c * CH
        pltpu.make_async_copy(feat.at[pl.ds(r0 * D, CH * D)], fb, semf).start()
        pltpu.make_async_copy(ids.at[pl.ds(r0, CH)], ib, semi).start()

    def wait(fb, ib, semf, semi):
        pltpu.make_async_copy(feat.at[pl.ds(0, CH * D)], fb, semf).wait()
        pltpu.make_async_copy(ids.at[pl.ds(0, CH)], ib, semi).wait()

    def finalize_run(pid, cnt, sxxv, s):
        @pl.when(cnt > 0)
        def _():
            cntv = cnt.astype(jnp.float32) + zero
            to_first = first_s[0] == 1

            @pl.when(to_first)
            def _():
                aux_v[pl.ds(0, L)] = pid.astype(jnp.float32) + zero
                aux_v[pl.ds(L, L)] = cntv
                aux_v[pl.ds(2 * L, L)] = sxxv
                for k in range(D // L):
                    recs_v[pl.ds(k * L, L)] = s[k]
                first_s[0] = 0

            @pl.when(jnp.logical_not(to_first))
            def _():
                sv = s[0] * s[0]
                for k in range(1, D // L):
                    sv = sv + s[k] * s[k]
                inv = 1.0 / cntv
                aux_v[pl.ds(6 * L, L)] = aux_v[pl.ds(6 * L, L)] + (
                    sxxv * inv - sv * inv * inv)
                aux_v[pl.ds(7 * L, L)] = aux_v[pl.ds(7 * L, L)] + 1.0

    def load_state():
        sxxv = st_v[pl.ds(0, L)]
        s = [st_v[pl.ds((1 + k) * L, L)] for k in range(D // L)]
        return sxxv, s

    def store_state(sxxv, s):
        st_v[pl.ds(0, L)] = sxxv
        for k in range(D // L):
            st_v[pl.ds((1 + k) * L, L)] = s[k]

    def rowload(fb, r):
        rb = r * D
        return [fb[pl.ds(rb + k * L, L)] for k in range(D // L)]

    def rowsq(x):
        p0 = x[0] * x[0] + x[1] * x[1]
        p1 = x[2] * x[2] + x[3] * x[3]
        p2 = x[4] * x[4] + x[5] * x[5]
        p3 = x[6] * x[6] + x[7] * x[7]
        return (p0 + p1) + (p2 + p3)

    def process(fb, ib, _):
        def grp(g, carry):
            idv = ib[pl.ds(g * L, L)]
            pid = pid_s[0]
            fast = idv[L - 1] == pid

            @pl.when(fast)
            def _():
                sxxv, s = load_state()
                for j in range(L):
                    x = rowload(fb, g * L + j)
                    sxxv = sxxv + rowsq(x)
                    s = [s[k] + x[k] for k in range(D // L)]
                store_state(sxxv, s)
                cnt_s[0] = cnt_s[0] + L

            @pl.when(jnp.logical_not(fast))
            def _():
                sxxv, s = load_state()
                p = pid
                cnt = cnt_s[0]
                for j in range(L):
                    rid = idv[j]
                    x = rowload(fb, g * L + j)
                    b = rid != p

                    @pl.when(b)
                    def _():
                        finalize_run(p, cnt, sxxv, s)

                    rq = rowsq(x)
                    sxxv = jnp.where(b, rq, sxxv + rq)
                    s = [jnp.where(b, x[k], s[k] + x[k])
                         for k in range(D // L)]
                    cnt = jnp.where(b, 1, cnt + 1)
                    p = rid
                store_state(sxxv, s)
                pid_s[0] = p
                cnt_s[0] = cnt

            return carry

        return lax.fori_loop(0, NG, grp, _)

    start(0, f0, i0, sf0, si0)

    def pair(p, carry):
        c0 = 2 * p
        start(c0 + 1, f1, i1, sf1, si1)
        wait(f0, i0, sf0, si0)
        carry = process(f0, i0, carry)
        start(c0 + 2, f0, i0, sf0, si0)
        wait(f1, i1, sf1, si1)
        carry = process(f1, i1, carry)
        return carry

    carry = lax.fori_loop(0, NCH // 2, pair, jnp.int32(0))
    wait(f0, i0, sf0, si0)
    process(f0, i0, carry)

    pid = pid_s[0]
    cnt = cnt_s[0]
    sxxv, s = load_state()
    cntv = cnt.astype(jnp.float32) + zero
    isf = first_s[0] == 1

    @pl.when(isf)
    def _():
        aux_v[pl.ds(0, L)] = pid.astype(jnp.float32) + zero
        aux_v[pl.ds(L, L)] = cntv
        aux_v[pl.ds(2 * L, L)] = sxxv
        aux_v[pl.ds(8 * L, L)] = 1.0 + zero
        for k in range(D // L):
            recs_v[pl.ds(k * L, L)] = s[k]

    @pl.when(jnp.logical_not(isf))
    def _():
        aux_v[pl.ds(3 * L, L)] = pid.astype(jnp.float32) + zero
        aux_v[pl.ds(4 * L, L)] = cntv
        aux_v[pl.ds(5 * L, L)] = sxxv
        for k in range(D // L):
            recs_v[pl.ds(D + k * L, L)] = s[k]

    pltpu.sync_copy(recs_v, rec_s.at[wid])
    pltpu.sync_copy(aux_v, aux.at[wid])


def _merge_body(recs_ref, aux_ref, out_ref):
    def step(w, carry):
        c_id, c_cnt, c_sxx, c_s, loss, uniq = carry
        av = aux_ref[pl.ds(w, 1)][0]
        rv = recs_ref[pl.ds(w, 1)][0]
        f_id = jnp.sum(av[0]) * (1.0 / 16.0)
        f_cnt = jnp.sum(av[1]) * (1.0 / 16.0)
        f_sxx = jnp.sum(av[2])
        l_id = jnp.sum(av[3]) * (1.0 / 16.0)
        l_cnt = jnp.sum(av[4]) * (1.0 / 16.0)
        l_sxx = jnp.sum(av[5])
        int_loss = jnp.sum(av[6])
        int_uniq = jnp.sum(av[7]) * (1.0 / 16.0)
        single = jnp.sum(av[8]) * (1.0 / 16.0) > 0.5
        f_s = rv[0:1, :]
        l_s = rv[1:2, :]

        mrg = jnp.logical_and(c_cnt > 0.0, c_id == f_id)
        fin = jnp.logical_and(jnp.logical_not(mrg), c_cnt > 0.0)
        safe = jnp.maximum(c_cnt, 1.0)
        c_ssq = jnp.sum(c_s * c_s)
        loss = loss + jnp.where(fin, c_sxx / safe - c_ssq / (safe * safe), 0.0)
        uniq = uniq + jnp.where(fin, 1.0, 0.0)

        m = jnp.where(mrg, 1.0, 0.0)
        f_cnt2 = f_cnt + m * c_cnt
        f_sxx2 = f_sxx + m * c_sxx
        f_s2 = f_s + m * c_s
        f_ssq = jnp.sum(f_s2 * f_s2)
        ffin = f_sxx2 / f_cnt2 - f_ssq / (f_cnt2 * f_cnt2)
        loss = loss + jnp.where(single, 0.0, ffin + int_loss)
        uniq = uniq + jnp.where(single, 0.0, 1.0 + int_uniq)

        c_id = jnp.where(single, f_id, l_id)
        c_cnt = jnp.where(single, f_cnt2, l_cnt)
        c_sxx = jnp.where(single, f_sxx2, l_sxx)
        c_s = jnp.where(single, f_s2, l_s)
        return (c_id, c_cnt, c_sxx, c_s, loss, uniq)

    init = (jnp.float32(-1.0), jnp.float32(0.0), jnp.float32(0.0),
            jnp.zeros((1, D), jnp.float32), jnp.float32(0.0), jnp.float32(0.0))
    c_id, c_cnt, c_sxx, c_s, loss, uniq = lax.fori_loop(0, NW, step, init)
    loss = loss + c_sxx / c_cnt - jnp.sum(c_s * c_s) / (c_cnt * c_cnt)
    uniq = uniq + 1.0
    out_ref[...] = jnp.full((1, 1), loss / uniq, jnp.float32)


@jax.jit
def kernel(reid_feat, ids):
    sc_phase = pl.kernel(
        _sc_body,
        out_type=(
            jax.ShapeDtypeStruct((NW, 2 * D), jnp.float32),
            jax.ShapeDtypeStruct((NW, 9 * L), jnp.float32),
        ),
        mesh=plsc.VectorSubcoreMesh(core_axis_name="c", subcore_axis_name="s",
                                    num_cores=NC, num_subcores=NS),
        scratch_types=[
            pltpu.VMEM((CH * D,), jnp.float32),
            pltpu.VMEM((CH * D,), jnp.float32),
            pltpu.VMEM((CH,), jnp.int32),
            pltpu.VMEM((CH,), jnp.int32),
            pltpu.VMEM((9 * L,), jnp.float32),
            pltpu.VMEM((2 * D,), jnp.float32),
            pltpu.VMEM((9 * L,), jnp.float32),
            pltpu.SMEM((1,), jnp.int32),
            pltpu.SMEM((1,), jnp.int32),
            pltpu.SMEM((1,), jnp.int32),
            pltpu.SemaphoreType.DMA,
            pltpu.SemaphoreType.DMA,
            pltpu.SemaphoreType.DMA,
            pltpu.SemaphoreType.DMA,
        ],
    )
    rec_s, aux = sc_phase(reid_feat.reshape(-1), ids.astype(jnp.int32))
    rec_s = rec_s.reshape(NW, 2, D)
    aux = aux.reshape(NW, 9, L)

    merged = pl.pallas_call(
        _merge_body,
        out_shape=jax.ShapeDtypeStruct((1, 1), jnp.float32),
        in_specs=[
            pl.BlockSpec(memory_space=pltpu.VMEM),
            pl.BlockSpec(memory_space=pltpu.VMEM),
        ],
        out_specs=pl.BlockSpec(memory_space=pltpu.VMEM),
    )(rec_s, aux)
    return merged[0, 0]


# speedup vs baseline: 1.9734x; 1.0027x over previous
import functools

import jax
import jax.numpy as jnp
from jax import lax
from jax.experimental import pallas as pl
from jax.experimental.pallas import tpu as pltpu
from jax.experimental.pallas import tpu_sc as plsc

N = 320000
D = 128
NC, NS, L = 2, 16, 16
NW = NC * NS
RPW = N // NW
CH = 400
NCH = RPW // CH
NG = CH // L


def _sc_body(feat, ids, rec_s, aux, f0, f1, i0, i1, st_v, recs_v, aux_v,
             first_s, pid_s, cnt_s, sf0, sf1, si0, si1):
    cid = lax.axis_index("c")
    sid = lax.axis_index("s")
    wid = cid * NS + sid
    base = wid * RPW

    zero = jnp.zeros((L,), jnp.float32)
    for k in range(2 * D // L):
        recs_v[pl.ds(k * L, L)] = zero
    for k in range(9):
        aux_v[pl.ds(k * L, L)] = zero
    for k in range(9):
        st_v[pl.ds(k * L, L)] = zero
    first_s[0] = 1
    pid_s[0] = -1
    cnt_s[0] = 0

    def start(c, fb, ib, semf, semi):
        r0 = base + c * CH
        pltpu.make_async_copy(feat.at[pl.ds(r0 * D, CH * D)], fb, semf).start()
        pltpu.make_async_copy(ids.at[pl.ds(r0, CH)], ib, semi).start()

    def wait(fb, ib, semf, semi):
        pltpu.make_async_copy(feat.at[pl.ds(0, CH * D)], fb, semf).wait()
        pltpu.make_async_copy(ids.at[pl.ds(0, CH)], ib, semi).wait()

    def finalize_run(pid, cnt, sxxv, s):
        @pl.when(cnt > 0)
        def _():
            cntv = cnt.astype(jnp.float32) + zero
            to_first = first_s[0] == 1

            @pl.when(to_first)
            def _():
                aux_v[pl.ds(0, L)] = pid.astype(jnp.float32) + zero
                aux_v[pl.ds(L, L)] = cntv
                aux_v[pl.ds(2 * L, L)] = sxxv
                for k in range(D // L):
                    recs_v[pl.ds(k * L, L)] = s[k]
                first_s[0] = 0

            @pl.when(jnp.logical_not(to_first))
            def _():
                sv = s[0] * s[0]
                for k in range(1, D // L):
                    sv = sv + s[k] * s[k]
                inv = 1.0 / cntv
                aux_v[pl.ds(6 * L, L)] = aux_v[pl.ds(6 * L, L)] + (
                    sxxv * inv - sv * inv * inv)
                aux_v[pl.ds(7 * L, L)] = aux_v[pl.ds(7 * L, L)] + 1.0

    def load_state():
        sxxv = st_v[pl.ds(0, L)]
        s = [st_v[pl.ds((1 + k) * L, L)] for k in range(D // L)]
        return sxxv, s

    def store_state(sxxv, s):
        st_v[pl.ds(0, L)] = sxxv
        for k in range(D // L):
            st_v[pl.ds((1 + k) * L, L)] = s[k]

    def rowload(fb, r):
        rb = r * D
        return [fb[pl.ds(rb + k * L, L)] for k in range(D // L)]

    def rowsq(x):
        p0 = x[0] * x[0] + x[1] * x[1]
        p1 = x[2] * x[2] + x[3] * x[3]
        p2 = x[4] * x[4] + x[5] * x[5]
        p3 = x[6] * x[6] + x[7] * x[7]
        return (p0 + p1) + (p2 + p3)

    def process(fb, ib, _):
        def grp(g, carry):
            idv = ib[pl.ds(g * L, L)]
            pid = pid_s[0]
            fast = idv[L - 1] == pid

            @pl.when(fast)
            def _():
                sxxv, s = load_state()
                sqp = [zero] * (D // L)
                for j in range(L):
                    x = rowload(fb, g * L + j)
                    s = [s[k] + x[k] for k in range(D // L)]
                    sqp = [sqp[k] + x[k] * x[k] for k in range(D // L)]
                q0 = (sqp[0] + sqp[1]) + (sqp[2] + sqp[3])
                q1 = (sqp[4] + sqp[5]) + (sqp[6] + sqp[7])
                sxxv = sxxv + (q0 + q1)
                store_state(sxxv, s)
                cnt_s[0] = cnt_s[0] + L

            @pl.when(jnp.logical_not(fast))
            def _():
                sxxv, s = load_state()
                p = pid
                cnt = cnt_s[0]
                for j in range(L):
                    rid = idv[j]
                    x = rowload(fb, g * L + j)
                    b = rid != p

                    @pl.when(b)
                    def _():
                        finalize_run(p, cnt, sxxv, s)

                    rq = rowsq(x)
                    sxxv = jnp.where(b, rq, sxxv + rq)
                    s = [jnp.where(b, x[k], s[k] + x[k])
                         for k in range(D // L)]
                    cnt = jnp.where(b, 1, cnt + 1)
                    p = rid
                store_state(sxxv, s)
                pid_s[0] = p
                cnt_s[0] = cnt

            return carry

        return lax.fori_loop(0, NG, grp, _)

    start(0, f0, i0, sf0, si0)

    def pair(p, carry):
        c0 = 2 * p
        start(c0 + 1, f1, i1, sf1, si1)
        wait(f0, i0, sf0, si0)
        carry = process(f0, i0, carry)
        start(c0 + 2, f0, i0, sf0, si0)
        wait(f1, i1, sf1, si1)
        carry = process(f1, i1, carry)
        return carry

    carry = lax.fori_loop(0, NCH // 2, pair, jnp.int32(0))
    wait(f0, i0, sf0, si0)
    process(f0, i0, carry)

    pid = pid_s[0]
    cnt = cnt_s[0]
    sxxv, s = load_state()
    cntv = cnt.astype(jnp.float32) + zero
    isf = first_s[0] == 1

    @pl.when(isf)
    def _():
        aux_v[pl.ds(0, L)] = pid.astype(jnp.float32) + zero
        aux_v[pl.ds(L, L)] = cntv
        aux_v[pl.ds(2 * L, L)] = sxxv
        aux_v[pl.ds(8 * L, L)] = 1.0 + zero
        for k in range(D // L):
            recs_v[pl.ds(k * L, L)] = s[k]

    @pl.when(jnp.logical_not(isf))
    def _():
        aux_v[pl.ds(3 * L, L)] = pid.astype(jnp.float32) + zero
        aux_v[pl.ds(4 * L, L)] = cntv
        aux_v[pl.ds(5 * L, L)] = sxxv
        for k in range(D // L):
            recs_v[pl.ds(D + k * L, L)] = s[k]

    pltpu.sync_copy(recs_v, rec_s.at[wid])
    pltpu.sync_copy(aux_v, aux.at[wid])


def _merge_body(recs_ref, aux_ref, out_ref):
    def step(w, carry):
        c_id, c_cnt, c_sxx, c_s, loss, uniq = carry
        av = aux_ref[pl.ds(w, 1)][0]
        rv = recs_ref[pl.ds(w, 1)][0]
        f_id = jnp.sum(av[0]) * (1.0 / 16.0)
        f_cnt = jnp.sum(av[1]) * (1.0 / 16.0)
        f_sxx = jnp.sum(av[2])
        l_id = jnp.sum(av[3]) * (1.0 / 16.0)
        l_cnt = jnp.sum(av[4]) * (1.0 / 16.0)
        l_sxx = jnp.sum(av[5])
        int_loss = jnp.sum(av[6])
        int_uniq = jnp.sum(av[7]) * (1.0 / 16.0)
        single = jnp.sum(av[8]) * (1.0 / 16.0) > 0.5
        f_s = rv[0:1, :]
        l_s = rv[1:2, :]

        mrg = jnp.logical_and(c_cnt > 0.0, c_id == f_id)
        fin = jnp.logical_and(jnp.logical_not(mrg), c_cnt > 0.0)
        safe = jnp.maximum(c_cnt, 1.0)
        c_ssq = jnp.sum(c_s * c_s)
        loss = loss + jnp.where(fin, c_sxx / safe - c_ssq / (safe * safe), 0.0)
        uniq = uniq + jnp.where(fin, 1.0, 0.0)

        m = jnp.where(mrg, 1.0, 0.0)
        f_cnt2 = f_cnt + m * c_cnt
        f_sxx2 = f_sxx + m * c_sxx
        f_s2 = f_s + m * c_s
        f_ssq = jnp.sum(f_s2 * f_s2)
        ffin = f_sxx2 / f_cnt2 - f_ssq / (f_cnt2 * f_cnt2)
        loss = loss + jnp.where(single, 0.0, ffin + int_loss)
        uniq = uniq + jnp.where(single, 0.0, 1.0 + int_uniq)

        c_id = jnp.where(single, f_id, l_id)
        c_cnt = jnp.where(single, f_cnt2, l_cnt)
        c_sxx = jnp.where(single, f_sxx2, l_sxx)
        c_s = jnp.where(single, f_s2, l_s)
        return (c_id, c_cnt, c_sxx, c_s, loss, uniq)

    init = (jnp.float32(-1.0), jnp.float32(0.0), jnp.float32(0.0),
            jnp.zeros((1, D), jnp.float32), jnp.float32(0.0), jnp.float32(0.0))
    c_id, c_cnt, c_sxx, c_s, loss, uniq = lax.fori_loop(0, NW, step, init)
    loss = loss + c_sxx / c_cnt - jnp.sum(c_s * c_s) / (c_cnt * c_cnt)
    uniq = uniq + 1.0
    out_ref[...] = jnp.full((1, 1), loss / uniq, jnp.float32)


@jax.jit
def kernel(reid_feat, ids):
    sc_phase = pl.kernel(
        _sc_body,
        out_type=(
            jax.ShapeDtypeStruct((NW, 2 * D), jnp.float32),
            jax.ShapeDtypeStruct((NW, 9 * L), jnp.float32),
        ),
        mesh=plsc.VectorSubcoreMesh(core_axis_name="c", subcore_axis_name="s",
                                    num_cores=NC, num_subcores=NS),
        scratch_types=[
            pltpu.VMEM((CH * D,), jnp.float32),
            pltpu.VMEM((CH * D,), jnp.float32),
            pltpu.VMEM((CH,), jnp.int32),
            pltpu.VMEM((CH,), jnp.int32),
            pltpu.VMEM((9 * L,), jnp.float32),
            pltpu.VMEM((2 * D,), jnp.float32),
            pltpu.VMEM((9 * L,), jnp.float32),
            pltpu.SMEM((1,), jnp.int32),
            pltpu.SMEM((1,), jnp.int32),
            pltpu.SMEM((1,), jnp.int32),
            pltpu.SemaphoreType.DMA,
            pltpu.SemaphoreType.DMA,
            pltpu.SemaphoreType.DMA,
            pltpu.SemaphoreType.DMA,
        ],
    )
    rec_s, aux = sc_phase(reid_feat.reshape(-1), ids.astype(jnp.int32))
    rec_s = rec_s.reshape(NW, 2, D)
    aux = aux.reshape(NW, 9, L)

    merged = pl.pallas_call(
        _merge_body,
        out_shape=jax.ShapeDtypeStruct((1, 1), jnp.float32),
        in_specs=[
            pl.BlockSpec(memory_space=pltpu.VMEM),
            pl.BlockSpec(memory_space=pltpu.VMEM),
        ],
        out_specs=pl.BlockSpec(memory_space=pltpu.VMEM),
    )(rec_s, aux)
    return merged[0, 0]
